# baseline (speedup 1.0000x reference)
import functools
import math

import jax
import jax.numpy as jnp
from jax import lax
from jax.experimental import pallas as pl
from jax.experimental.pallas import tpu as pltpu
from jax.experimental.pallas import tpu_sc as plsc

N = 10000
D = 128
E = 320000

NC = 2
NS = 16
NW = NC * NS
EPW = E // NW
B = 80
NBLK = EPW // B
BA = 48
BB = 32
NPD = 10240
RPS = NPD // NS
DPS = NPD // NS

_SQRT_HALF = 1.0 / math.sqrt(2.0)


def _gelu(x):
    return 0.5 * x * (1.0 + lax.erf(x * _SQRT_HALF))


def _qkv_body(x_ref, wkqv_ref, bkqv_ref, wk_ref, wv_ref, p_ref,
              q_ref, k_ref, v_ref):
    x = x_ref[...]
    kqv = jnp.dot(x, wkqv_ref[...], preferred_element_type=jnp.float32)
    kqv = kqv + bkqv_ref[...]
    k0 = kqv[:, 0:D]
    q0 = kqv[:, D:2 * D]
    v0 = kqv[:, 2 * D:3 * D]
    k_ref[...] = jnp.dot(k0, wk_ref[...], preferred_element_type=jnp.float32)
    v_ref[...] = jnp.dot(v0, wv_ref[...], preferred_element_type=jnp.float32)
    q_ref[...] = q0 * (p_ref[0, 0] * (1.0 / math.sqrt(float(D))))


def _qkv_call(x, wkqv, bkqv, wk, wv, p):
    nb = 10
    rb = N // nb
    full = lambda shape: pl.BlockSpec(shape, lambda i: (0, 0))
    row = pl.BlockSpec((rb, D), lambda i: (i, 0))
    return pl.pallas_call(
        _qkv_body,
        grid=(nb,),
        in_specs=[row, full((D, 3 * D)), full((1, 3 * D)), full((D, D)),
                  full((D, D)), full((1, 1))],
        out_specs=[row, row, row],
        out_shape=[jax.ShapeDtypeStruct((N, D), jnp.float32)] * 3,
    )(x, wkqv, bkqv, wk, wv, p)


def _edge_body(q_hbm, k_hbm, v_hbm, src_hbm, dst_hbm, agg_out, den_out,
               srcAB, dstAB, qrA, krA, vrA, exvA,
               qrB, krB, vrB, exvB,
               zrow, zden, agg_sh, den_sh,
               isem, gsemA, gsemB, ssem):
    c = lax.axis_index("c")
    s = lax.axis_index("s")
    wid = c * NS + s

    zv = jnp.zeros((16,), jnp.float32)
    for r in range(16):
        for ch in range(8):
            zrow[r, pl.ds(ch * 16, 16)] = zv
    for i in range(DPS // 16):
        zden[pl.ds(i * 16, 16)] = zv
    for j in range(RPS // 16):
        pltpu.sync_copy(zrow, agg_sh.at[pl.ds(pl.multiple_of(s * RPS + j * 16, 8), 16)])
    pltpu.sync_copy(zden, den_sh.at[pl.ds(pl.multiple_of(s * DPS, 8), DPS)])
    plsc.subcore_barrier()

    base0 = wid * EPW
    lane = lax.iota(jnp.int32, 16)

    def compute(qr, kr, vr, exv, nsub):
        for grp in range(nsub // 16):
            av = jnp.zeros((16,), jnp.float32)
            for j in range(16):
                e = grp * 16 + j
                acc = qr[e, pl.ds(0, 16)] * kr[e, pl.ds(0, 16)]
                for ch in range(1, 8):
                    acc = acc + qr[e, pl.ds(ch * 16, 16)] * kr[e, pl.ds(ch * 16, 16)]
                av = jnp.where(lane == j, jnp.sum(acc), av)
            exa = jnp.exp(av)
            exv[pl.ds(grp * 16, 16)] = exa
            for j in range(16):
                e = grp * 16 + j
                sc = jnp.sum(jnp.where(lane == j, exa, 0.0))
                for ch in range(8):
                    vr[e, pl.ds(ch * 16, 16)] = vr[e, pl.ds(ch * 16, 16)] * sc

    def blk_body(i, carry):
        base = pl.multiple_of(base0 + i * B, 8)
        pltpu.async_copy(src_hbm.at[pl.ds(base, B)], srcAB, isem)
        pltpu.async_copy(dst_hbm.at[pl.ds(base, B)], dstAB, isem)
        pltpu.make_async_copy(src_hbm.at[pl.ds(base, B)], srcAB, isem).wait()
        pltpu.make_async_copy(dst_hbm.at[pl.ds(base, B)], dstAB, isem).wait()
        sA = srcAB.at[pl.ds(0, BA)]
        dA = dstAB.at[pl.ds(0, BA)]
        sB = srcAB.at[pl.ds(pl.multiple_of(BA, 8), BB)]
        dB = dstAB.at[pl.ds(pl.multiple_of(BA, 8), BB)]
        pltpu.async_copy(q_hbm.at[dA], qrA, gsemA)
        pltpu.async_copy(k_hbm.at[sA], krA, gsemA)
        pltpu.async_copy(v_hbm.at[sA], vrA, gsemA)
        pltpu.async_copy(q_hbm.at[dB], qrB, gsemB)
        pltpu.async_copy(k_hbm.at[sB], krB, gsemB)
        pltpu.async_copy(v_hbm.at[sB], vrB, gsemB)
        pltpu.make_async_copy(q_hbm.at[dA], qrA, gsemA).wait()
        pltpu.make_async_copy(k_hbm.at[sA], krA, gsemA).wait()
        pltpu.make_async_copy(v_hbm.at[sA], vrA, gsemA).wait()
        compute(qrA, krA, vrA, exvA, BA)
        pltpu.async_copy(vrA, agg_sh.at[dA], ssem, add=True)
        pltpu.async_copy(exvA, den_sh.at[dA], ssem, add=True)
        pltpu.make_async_copy(q_hbm.at[dB], qrB, gsemB).wait()
        pltpu.make_async_copy(k_hbm.at[sB], krB, gsemB).wait()
        pltpu.make_async_copy(v_hbm.at[sB], vrB, gsemB).wait()
        compute(qrB, krB, vrB, exvB, BB)
        pltpu.async_copy(vrB, agg_sh.at[dB], ssem, add=True)
        pltpu.async_copy(exvB, den_sh.at[dB], ssem, add=True)
        pltpu.make_async_copy(vrA, agg_sh.at[dA], ssem).wait()
        pltpu.make_async_copy(exvA, den_sh.at[dA], ssem).wait()
        pltpu.make_async_copy(vrB, agg_sh.at[dB], ssem).wait()
        pltpu.make_async_copy(exvB, den_sh.at[dB], ssem).wait()
        return carry

    lax.fori_loop(0, NBLK, blk_body, 0)
    plsc.subcore_barrier()

    pltpu.sync_copy(agg_sh.at[pl.ds(pl.multiple_of(s * RPS, 8), RPS)],
                    agg_out.at[pl.ds(pl.multiple_of(c * NPD + s * RPS, 8), RPS)])
    pltpu.sync_copy(den_sh.at[pl.ds(pl.multiple_of(s * DPS, 8), DPS)],
                    den_out.at[pl.ds(pl.multiple_of(c * NPD + s * DPS, 8), DPS)])


_edge_call = functools.partial(
    pl.kernel,
    out_type=[jax.ShapeDtypeStruct((2 * NPD, D), jnp.float32),
              jax.ShapeDtypeStruct((2 * NPD,), jnp.float32)],
    mesh=plsc.VectorSubcoreMesh(core_axis_name="c", subcore_axis_name="s"),
    compiler_params=pltpu.CompilerParams(needs_layout_passes=False),
    scratch_types=[
        pltpu.VMEM((B,), jnp.int32),
        pltpu.VMEM((B,), jnp.int32),
        pltpu.VMEM((BA, D), jnp.float32),
        pltpu.VMEM((BA, D), jnp.float32),
        pltpu.VMEM((BA, D), jnp.float32),
        pltpu.VMEM((BA,), jnp.float32),
        pltpu.VMEM((BB, D), jnp.float32),
        pltpu.VMEM((BB, D), jnp.float32),
        pltpu.VMEM((BB, D), jnp.float32),
        pltpu.VMEM((BB,), jnp.float32),
        pltpu.VMEM((16, D), jnp.float32),
        pltpu.VMEM((DPS,), jnp.float32),
        pltpu.VMEM_SHARED((NPD, D), jnp.float32),
        pltpu.VMEM_SHARED((NPD,), jnp.float32),
        pltpu.SemaphoreType.DMA,
        pltpu.SemaphoreType.DMA,
        pltpu.SemaphoreType.DMA,
        pltpu.SemaphoreType.DMA,
    ],
)(_edge_body)


def _combine_core(agg0_ref, agg1_ref, den_ref, xin_ref, wo_ref, bo_ref,
                  skip_ref):
    num = agg0_ref[0] + agg1_ref[0]
    den = den_ref[0, 0] + den_ref[0, 1] + 1e-16
    aggn = num * (1.0 / den)[:, None]
    out = jnp.dot(_gelu(aggn), wo_ref[...], preferred_element_type=jnp.float32)
    out = out + bo_ref[...]
    a = jax.nn.sigmoid(skip_ref[0, 0])
    return a * out + (1.0 - a) * xin_ref[...]


def _combine_qkv_body(agg0_ref, agg1_ref, den_ref, xin_ref, wo_ref, bo_ref,
                      skip_ref, wkqv_ref, bkqv_ref, wk_ref, wv_ref, p_ref,
                      h_ref, q_ref, k_ref, v_ref):
    h = _combine_core(agg0_ref, agg1_ref, den_ref, xin_ref, wo_ref, bo_ref,
                      skip_ref)
    h_ref[...] = h
    kqv = jnp.dot(h, wkqv_ref[...], preferred_element_type=jnp.float32)
    kqv = kqv + bkqv_ref[...]
    k_ref[...] = jnp.dot(kqv[:, 0:D], wk_ref[...],
                         preferred_element_type=jnp.float32)
    v_ref[...] = jnp.dot(kqv[:, 2 * D:3 * D], wv_ref[...],
                         preferred_element_type=jnp.float32)
    q_ref[...] = kqv[:, D:2 * D] * (p_ref[0, 0] * (1.0 / math.sqrt(float(D))))


def _final_body(agg0_ref, agg1_ref, den_ref, xin_ref, wo_ref, bo_ref,
                skip_ref, wlin_ref, blin_ref, o_ref):
    h = _combine_core(agg0_ref, agg1_ref, den_ref, xin_ref, wo_ref, bo_ref,
                      skip_ref)
    o_ref[...] = jnp.dot(h, wlin_ref[...],
                         preferred_element_type=jnp.float32) + blin_ref[...]


def _den3d(denf):
    nb = 10
    rb = N // nb
    d = denf.reshape(2, NPD)[:, :N].reshape(2, nb, rb)
    return d.transpose(1, 0, 2)


def _combine_qkv_call(aggf, denf, xin, wo, bo, skip, wkqv, bkqv, wk, wv, p):
    nb = 10
    rb = N // nb
    full = lambda shape: pl.BlockSpec(shape, lambda i: tuple(0 for _ in shape))
    row = pl.BlockSpec((rb, D), lambda i: (i, 0))
    a0 = pl.BlockSpec((1, rb, D), lambda i: (0, i, 0))
    a1 = pl.BlockSpec((1, rb, D), lambda i: (1, i, 0))
    dsp = pl.BlockSpec((1, 2, rb), lambda i: (i, 0, 0))
    return pl.pallas_call(
        _combine_qkv_body,
        grid=(nb,),
        in_specs=[a0, a1, dsp, row, full((D, D)), full((1, D)), full((1, 1)),
                  full((D, 3 * D)), full((1, 3 * D)), full((D, D)),
                  full((D, D)), full((1, 1))],
        out_specs=[row, row, row, row],
        out_shape=[jax.ShapeDtypeStruct((N, D), jnp.float32)] * 4,
    )(aggf.reshape(2, NPD, D), aggf.reshape(2, NPD, D), _den3d(denf),
      xin, wo, bo, skip, wkqv, bkqv, wk, wv, p)


def _final_call(aggf, denf, xin, wo, bo, skip, wlin_pad, blin_pad):
    nb = 10
    rb = N // nb
    full = lambda shape: pl.BlockSpec(shape, lambda i: tuple(0 for _ in shape))
    row = pl.BlockSpec((rb, D), lambda i: (i, 0))
    a0 = pl.BlockSpec((1, rb, D), lambda i: (0, i, 0))
    a1 = pl.BlockSpec((1, rb, D), lambda i: (1, i, 0))
    dsp = pl.BlockSpec((1, 2, rb), lambda i: (i, 0, 0))
    return pl.pallas_call(
        _final_body,
        grid=(nb,),
        in_specs=[a0, a1, dsp, row, full((D, D)), full((1, D)), full((1, 1)),
                  full((D, D)), full((1, D))],
        out_specs=row,
        out_shape=jax.ShapeDtypeStruct((N, D), jnp.float32),
    )(aggf.reshape(2, NPD, D), aggf.reshape(2, NPD, D), _den3d(denf),
      xin, wo, bo, skip, wlin_pad, blin_pad)


def kernel(x, edge_index,
           W_kqv1, b_kqv1, Wk1, Wv1, p1, Wo1, bo1, skip1,
           W_kqv2, b_kqv2, Wk2, Wv2, p2, Wo2, bo2, skip2,
           W_lin, b_lin):
    src = edge_index[0]
    dst = edge_index[1]
    b1 = b_kqv1.reshape(1, 3 * D)
    b2 = b_kqv2.reshape(1, 3 * D)
    bo1r = bo1.reshape(1, D)
    bo2r = bo2.reshape(1, D)
    s1 = skip1.reshape(1, 1)
    s2 = skip2.reshape(1, 1)
    wlin_pad = jnp.pad(W_lin, ((0, 0), (0, D - W_lin.shape[1])))
    blin_pad = jnp.pad(b_lin, (0, D - b_lin.shape[0])).reshape(1, D)

    q1, k1, v1 = _qkv_call(x, W_kqv1, b1, Wk1, Wv1, p1)
    aggf1, denf1 = _edge_call(q1, k1, v1, src, dst)
    h1, q2, k2, v2 = _combine_qkv_call(aggf1, denf1, x, Wo1, bo1r, s1,
                                       W_kqv2, b2, Wk2, Wv2, p2)
    aggf2, denf2 = _edge_call(q2, k2, v2, src, dst)
    outp = _final_call(aggf2, denf2, h1, Wo2, bo2r, s2, wlin_pad, blin_pad)
    return outp[:, :W_lin.shape[1]]


# speedup vs baseline: 1.0225x; 1.0225x over previous
import functools
import math

import jax
import jax.numpy as jnp
from jax import lax
from jax.experimental import pallas as pl
from jax.experimental.pallas import tpu as pltpu
from jax.experimental.pallas import tpu_sc as plsc

N = 10000
D = 128
E = 320000

NC = 2
NS = 16
NW = NC * NS
EPW = E // NW
B = 80
NBLK = EPW // B
BA = 48
BB = 32
NPD = 10240
RPS = NPD // NS
DPS = NPD // NS

_SQRT_HALF = 1.0 / math.sqrt(2.0)


def _gelu(x):
    return 0.5 * x * (1.0 + lax.erf(x * _SQRT_HALF))


def _qkv_body(x_ref, wkqv_ref, bkqv_ref, wk_ref, wv_ref, p_ref,
              q_ref, k_ref, v_ref):
    x = x_ref[...]
    kqv = jnp.dot(x, wkqv_ref[...], preferred_element_type=jnp.float32)
    kqv = kqv + bkqv_ref[...]
    k0 = kqv[:, 0:D]
    q0 = kqv[:, D:2 * D]
    v0 = kqv[:, 2 * D:3 * D]
    k_ref[...] = jnp.dot(k0, wk_ref[...], preferred_element_type=jnp.float32)
    v_ref[...] = jnp.dot(v0, wv_ref[...], preferred_element_type=jnp.float32)
    q_ref[...] = q0 * (p_ref[0, 0] * (1.0 / math.sqrt(float(D))))


def _qkv_call(x, wkqv, bkqv, wk, wv, p):
    nb = 10
    rb = N // nb
    full = lambda shape: pl.BlockSpec(shape, lambda i: (0, 0))
    row = pl.BlockSpec((rb, D), lambda i: (i, 0))
    return pl.pallas_call(
        _qkv_body,
        grid=(nb,),
        in_specs=[row, full((D, 3 * D)), full((1, 3 * D)), full((D, D)),
                  full((D, D)), full((1, 1))],
        out_specs=[row, row, row],
        out_shape=[jax.ShapeDtypeStruct((N, D), jnp.float32)] * 3,
    )(x, wkqv, bkqv, wk, wv, p)


def _edge_body(q_hbm, k_hbm, v_hbm, src_hbm, dst_hbm, agg_out, den_out,
               srcA, dstA, dstsA, qrA, krA, vrA, exvA,
               srcB, dstB, dstsB, qrB, krB, vrB, exvB,
               zrow, zden, agg_sh, den_sh,
               isemA, isemB, gsemA, gsemB, ssem):
    c = lax.axis_index("c")
    s = lax.axis_index("s")
    wid = c * NS + s

    zv = jnp.zeros((16,), jnp.float32)
    for r in range(16):
        for ch in range(8):
            zrow[r, pl.ds(ch * 16, 16)] = zv
    for i in range(DPS // 16):
        zden[pl.ds(i * 16, 16)] = zv
    for j in range(RPS // 16):
        pltpu.sync_copy(zrow, agg_sh.at[pl.ds(pl.multiple_of(s * RPS + j * 16, 8), 16)])
    pltpu.sync_copy(zden, den_sh.at[pl.ds(pl.multiple_of(s * DPS, 8), DPS)])
    plsc.subcore_barrier()

    base0 = wid * EPW
    lane = lax.iota(jnp.int32, 16)

    def compute(qr, kr, vr, exv, nsub):
        for grp in range(nsub // 16):
            av = jnp.zeros((16,), jnp.float32)
            for j in range(16):
                e = grp * 16 + j
                acc = qr[e, pl.ds(0, 16)] * kr[e, pl.ds(0, 16)]
                for ch in range(1, 8):
                    acc = acc + qr[e, pl.ds(ch * 16, 16)] * kr[e, pl.ds(ch * 16, 16)]
                av = jnp.where(lane == j, jnp.sum(acc), av)
            exa = jnp.exp(av)
            exv[pl.ds(grp * 16, 16)] = exa
            for j in range(16):
                e = grp * 16 + j
                sc = jnp.sum(jnp.where(lane == j, exa, 0.0))
                for ch in range(8):
                    vr[e, pl.ds(ch * 16, 16)] = vr[e, pl.ds(ch * 16, 16)] * sc

    def blk_body(i, carry):
        baseA = pl.multiple_of(base0 + i * B, 8)
        baseB = pl.multiple_of(base0 + i * B + BA, 8)
        pltpu.async_copy(src_hbm.at[pl.ds(baseA, BA)], srcA, isemA)
        pltpu.async_copy(dst_hbm.at[pl.ds(baseA, BA)], dstA, isemA)
        pltpu.async_copy(dst_hbm.at[pl.ds(baseA, BA)], dstsA, isemA)
        pltpu.async_copy(src_hbm.at[pl.ds(baseB, BB)], srcB, isemB)
        pltpu.async_copy(dst_hbm.at[pl.ds(baseB, BB)], dstB, isemB)
        pltpu.async_copy(dst_hbm.at[pl.ds(baseB, BB)], dstsB, isemB)
        pltpu.make_async_copy(src_hbm.at[pl.ds(baseA, BA)], srcA, isemA).wait()
        pltpu.make_async_copy(dst_hbm.at[pl.ds(baseA, BA)], dstA, isemA).wait()
        pltpu.make_async_copy(dst_hbm.at[pl.ds(baseA, BA)], dstsA, isemA).wait()
        pltpu.async_copy(q_hbm.at[dstA], qrA, gsemA)
        pltpu.async_copy(k_hbm.at[srcA], krA, gsemA)
        pltpu.async_copy(v_hbm.at[srcA], vrA, gsemA)
        pltpu.make_async_copy(src_hbm.at[pl.ds(baseB, BB)], srcB, isemB).wait()
        pltpu.make_async_copy(dst_hbm.at[pl.ds(baseB, BB)], dstB, isemB).wait()
        pltpu.make_async_copy(dst_hbm.at[pl.ds(baseB, BB)], dstsB, isemB).wait()
        pltpu.async_copy(q_hbm.at[dstB], qrB, gsemB)
        pltpu.async_copy(k_hbm.at[srcB], krB, gsemB)
        pltpu.async_copy(v_hbm.at[srcB], vrB, gsemB)
        pltpu.make_async_copy(q_hbm.at[dstA], qrA, gsemA).wait()
        pltpu.make_async_copy(k_hbm.at[srcA], krA, gsemA).wait()
        pltpu.make_async_copy(v_hbm.at[srcA], vrA, gsemA).wait()
        compute(qrA, krA, vrA, exvA, BA)
        pltpu.async_copy(vrA, agg_sh.at[dstsA], ssem, add=True)
        pltpu.async_copy(exvA, den_sh.at[dstsA], ssem, add=True)
        pltpu.make_async_copy(q_hbm.at[dstB], qrB, gsemB).wait()
        pltpu.make_async_copy(k_hbm.at[srcB], krB, gsemB).wait()
        pltpu.make_async_copy(v_hbm.at[srcB], vrB, gsemB).wait()
        compute(qrB, krB, vrB, exvB, BB)
        pltpu.async_copy(vrB, agg_sh.at[dstsB], ssem, add=True)
        pltpu.async_copy(exvB, den_sh.at[dstsB], ssem, add=True)
        pltpu.make_async_copy(vrA, agg_sh.at[dstsA], ssem).wait()
        pltpu.make_async_copy(exvA, den_sh.at[dstsA], ssem).wait()
        pltpu.make_async_copy(vrB, agg_sh.at[dstsB], ssem).wait()
        pltpu.make_async_copy(exvB, den_sh.at[dstsB], ssem).wait()
        return carry

    lax.fori_loop(0, NBLK, blk_body, 0)
    plsc.subcore_barrier()

    pltpu.sync_copy(agg_sh.at[pl.ds(pl.multiple_of(s * RPS, 8), RPS)],
                    agg_out.at[pl.ds(pl.multiple_of(c * NPD + s * RPS, 8), RPS)])
    pltpu.sync_copy(den_sh.at[pl.ds(pl.multiple_of(s * DPS, 8), DPS)],
                    den_out.at[pl.ds(pl.multiple_of(c * NPD + s * DPS, 8), DPS)])


_edge_call = functools.partial(
    pl.kernel,
    out_type=[jax.ShapeDtypeStruct((2 * NPD, D), jnp.float32),
              jax.ShapeDtypeStruct((2 * NPD,), jnp.float32)],
    mesh=plsc.VectorSubcoreMesh(core_axis_name="c", subcore_axis_name="s"),
    compiler_params=pltpu.CompilerParams(needs_layout_passes=False),
    scratch_types=[
        pltpu.VMEM((BA,), jnp.int32),
        pltpu.VMEM((BA,), jnp.int32),
        pltpu.VMEM((BA,), jnp.int32),
        pltpu.VMEM((BA, D), jnp.float32),
        pltpu.VMEM((BA, D), jnp.float32),
        pltpu.VMEM((BA, D), jnp.float32),
        pltpu.VMEM((BA,), jnp.float32),
        pltpu.VMEM((BB,), jnp.int32),
        pltpu.VMEM((BB,), jnp.int32),
        pltpu.VMEM((BB,), jnp.int32),
        pltpu.VMEM((BB, D), jnp.float32),
        pltpu.VMEM((BB, D), jnp.float32),
        pltpu.VMEM((BB, D), jnp.float32),
        pltpu.VMEM((BB,), jnp.float32),
        pltpu.VMEM((16, D), jnp.float32),
        pltpu.VMEM((DPS,), jnp.float32),
        pltpu.VMEM_SHARED((NPD, D), jnp.float32),
        pltpu.VMEM_SHARED((NPD,), jnp.float32),
        pltpu.SemaphoreType.DMA,
        pltpu.SemaphoreType.DMA,
        pltpu.SemaphoreType.DMA,
        pltpu.SemaphoreType.DMA,
        pltpu.SemaphoreType.DMA,
    ],
)(_edge_body)


def _combine_core(agg0_ref, agg1_ref, den_ref, xin_ref, wo_ref, bo_ref,
                  skip_ref):
    num = agg0_ref[0] + agg1_ref[0]
    den = den_ref[0, 0] + den_ref[0, 1] + 1e-16
    aggn = num * (1.0 / den)[:, None]
    out = jnp.dot(_gelu(aggn), wo_ref[...], preferred_element_type=jnp.float32)
    out = out + bo_ref[...]
    a = jax.nn.sigmoid(skip_ref[0, 0])
    return a * out + (1.0 - a) * xin_ref[...]


def _combine_qkv_body(agg0_ref, agg1_ref, den_ref, xin_ref, wo_ref, bo_ref,
                      skip_ref, wkqv_ref, bkqv_ref, wk_ref, wv_ref, p_ref,
                      h_ref, q_ref, k_ref, v_ref):
    h = _combine_core(agg0_ref, agg1_ref, den_ref, xin_ref, wo_ref, bo_ref,
                      skip_ref)
    h_ref[...] = h
    kqv = jnp.dot(h, wkqv_ref[...], preferred_element_type=jnp.float32)
    kqv = kqv + bkqv_ref[...]
    k_ref[...] = jnp.dot(kqv[:, 0:D], wk_ref[...],
                         preferred_element_type=jnp.float32)
    v_ref[...] = jnp.dot(kqv[:, 2 * D:3 * D], wv_ref[...],
                         preferred_element_type=jnp.float32)
    q_ref[...] = kqv[:, D:2 * D] * (p_ref[0, 0] * (1.0 / math.sqrt(float(D))))


def _final_body(agg0_ref, agg1_ref, den_ref, xin_ref, wo_ref, bo_ref,
                skip_ref, wlin_ref, blin_ref, o_ref):
    h = _combine_core(agg0_ref, agg1_ref, den_ref, xin_ref, wo_ref, bo_ref,
                      skip_ref)
    o_ref[...] = jnp.dot(h, wlin_ref[...],
                         preferred_element_type=jnp.float32) + blin_ref[...]


def _den3d(denf):
    nb = 10
    rb = N // nb
    d = denf.reshape(2, NPD)[:, :N].reshape(2, nb, rb)
    return d.transpose(1, 0, 2)


def _combine_qkv_call(aggf, denf, xin, wo, bo, skip, wkqv, bkqv, wk, wv, p):
    nb = 10
    rb = N // nb
    full = lambda shape: pl.BlockSpec(shape, lambda i: tuple(0 for _ in shape))
    row = pl.BlockSpec((rb, D), lambda i: (i, 0))
    a0 = pl.BlockSpec((1, rb, D), lambda i: (0, i, 0))
    a1 = pl.BlockSpec((1, rb, D), lambda i: (1, i, 0))
    dsp = pl.BlockSpec((1, 2, rb), lambda i: (i, 0, 0))
    return pl.pallas_call(
        _combine_qkv_body,
        grid=(nb,),
        in_specs=[a0, a1, dsp, row, full((D, D)), full((1, D)), full((1, 1)),
                  full((D, 3 * D)), full((1, 3 * D)), full((D, D)),
                  full((D, D)), full((1, 1))],
        out_specs=[row, row, row, row],
        out_shape=[jax.ShapeDtypeStruct((N, D), jnp.float32)] * 4,
    )(aggf.reshape(2, NPD, D), aggf.reshape(2, NPD, D), _den3d(denf),
      xin, wo, bo, skip, wkqv, bkqv, wk, wv, p)


def _final_call(aggf, denf, xin, wo, bo, skip, wlin_pad, blin_pad):
    nb = 10
    rb = N // nb
    full = lambda shape: pl.BlockSpec(shape, lambda i: tuple(0 for _ in shape))
    row = pl.BlockSpec((rb, D), lambda i: (i, 0))
    a0 = pl.BlockSpec((1, rb, D), lambda i: (0, i, 0))
    a1 = pl.BlockSpec((1, rb, D), lambda i: (1, i, 0))
    dsp = pl.BlockSpec((1, 2, rb), lambda i: (i, 0, 0))
    return pl.pallas_call(
        _final_body,
        grid=(nb,),
        in_specs=[a0, a1, dsp, row, full((D, D)), full((1, D)), full((1, 1)),
                  full((D, D)), full((1, D))],
        out_specs=row,
        out_shape=jax.ShapeDtypeStruct((N, D), jnp.float32),
    )(aggf.reshape(2, NPD, D), aggf.reshape(2, NPD, D), _den3d(denf),
      xin, wo, bo, skip, wlin_pad, blin_pad)


def kernel(x, edge_index,
           W_kqv1, b_kqv1, Wk1, Wv1, p1, Wo1, bo1, skip1,
           W_kqv2, b_kqv2, Wk2, Wv2, p2, Wo2, bo2, skip2,
           W_lin, b_lin):
    src = edge_index[0]
    dst = edge_index[1]
    b1 = b_kqv1.reshape(1, 3 * D)
    b2 = b_kqv2.reshape(1, 3 * D)
    bo1r = bo1.reshape(1, D)
    bo2r = bo2.reshape(1, D)
    s1 = skip1.reshape(1, 1)
    s2 = skip2.reshape(1, 1)
    wlin_pad = jnp.pad(W_lin, ((0, 0), (0, D - W_lin.shape[1])))
    blin_pad = jnp.pad(b_lin, (0, D - b_lin.shape[0])).reshape(1, D)

    q1, k1, v1 = _qkv_call(x, W_kqv1, b1, Wk1, Wv1, p1)
    aggf1, denf1 = _edge_call(q1, k1, v1, src, dst)
    h1, q2, k2, v2 = _combine_qkv_call(aggf1, denf1, x, Wo1, bo1r, s1,
                                       W_kqv2, b2, Wk2, Wv2, p2)
    aggf2, denf2 = _edge_call(q2, k2, v2, src, dst)
    outp = _final_call(aggf2, denf2, h1, Wo2, bo2r, s2, wlin_pad, blin_pad)
    return outp[:, :W_lin.shape[1]]
